# baseline (speedup 1.0000x reference)
import dataclasses
import functools

import jax
import jax.numpy as jnp
from jax import lax
from jax.experimental import pallas as pl
from jax.experimental.pallas import tpu as pltpu
from jax.experimental.pallas import tpu_sc as plsc

B = 4096
D = 32
NUM_WORKERS = 32
B_PER_W = B // NUM_WORKERS
LANES = 16
W = 128
CH = 16
N_CH = B_PER_W // CH


def _sc_params():
  cp = pltpu.CompilerParams()
  if "needs_layout_passes" in pltpu.CompilerParams.__dataclass_fields__:
    cp = dataclasses.replace(cp, needs_layout_passes=False)
  if "use_tc_tiling_on_sc" in pltpu.CompilerParams.__dataclass_fields__:
    cp = dataclasses.replace(cp, use_tc_tiling_on_sc=True)
  return cp


def _mesh():
  return plsc.VectorSubcoreMesh(core_axis_name="c", subcore_axis_name="s")


def _sc_embed_dot(word_u, word_v, in_embed_t, out_embed_t):
  @functools.partial(
      pl.kernel,
      compiler_params=_sc_params(),
      out_type=jax.ShapeDtypeStruct((B,), jnp.float32),
      mesh=_mesh(),
      scratch_types=[
          pltpu.VMEM((B_PER_W,), jnp.int32),
          pltpu.VMEM((B_PER_W,), jnp.int32),
          pltpu.VMEM((CH, D, W), jnp.float32),
          pltpu.VMEM((D, CH), jnp.float32),
          pltpu.VMEM((B_PER_W,), jnp.float32),
          pltpu.SemaphoreType.DMA,
          pltpu.SemaphoreType.DMA,
      ],
  )
  def k(word_u_hbm, word_v_hbm, u_tab_hbm, v_tab_hbm, dots_hbm,
        idx_u_v, idx_v_v, blk, u_comp, dots_v, sem, bsem):
    wid = lax.axis_index("s") * 2 + lax.axis_index("c")
    base = wid * B_PER_W

    c1 = pltpu.async_copy(word_u_hbm.at[pl.ds(base, B_PER_W)], idx_u_v, bsem)
    c2 = pltpu.async_copy(word_v_hbm.at[pl.ds(base, B_PER_W)], idx_v_v, bsem)
    c1.wait()
    c2.wait()

    iota = lax.iota(jnp.int32, LANES)
    for c in range(N_CH):
      off = c * CH
      lu = idx_u_v[pl.ds(off, LANES)]
      lv = idx_v_v[pl.ds(off, LANES)]
      jb_u = (lu // W) * W
      jb_v = (lv // W) * W
      cops = []
      for i in range(CH):
        ju = jnp.sum(jnp.where(iota == i, jb_u, 0))
        cops.append(pltpu.async_copy(
            u_tab_hbm.at[:, pl.ds(pl.multiple_of(ju, W), W)], blk.at[i], sem))
      for cp_ in cops:
        cp_.wait()
      lane_u = lu - jb_u
      for d in range(D):
        dd = jnp.full((LANES,), d, jnp.int32)
        u_comp[d, :] = plsc.load_gather(blk, [iota, dd, lane_u])
      cops = []
      for i in range(CH):
        jv = jnp.sum(jnp.where(iota == i, jb_v, 0))
        cops.append(pltpu.async_copy(
            v_tab_hbm.at[:, pl.ds(pl.multiple_of(jv, W), W)], blk.at[i], sem))
      for cp_ in cops:
        cp_.wait()
      lane_v = lv - jb_v
      acc = jnp.zeros((LANES,), jnp.float32)
      for d in range(D):
        dd = jnp.full((LANES,), d, jnp.int32)
        vv = plsc.load_gather(blk, [iota, dd, lane_v])
        acc = acc + u_comp[d, :] * vv
      dots_v[pl.ds(off, LANES)] = acc

    s1 = pltpu.async_copy(dots_v, dots_hbm.at[pl.ds(base, B_PER_W)], bsem)
    s1.wait()

  return k(word_u, word_v, in_embed_t, out_embed_t)


def _sc_bias_sum(word_u, word_v, in_bias_flat, out_bias_flat):
  @functools.partial(
      pl.kernel,
      compiler_params=_sc_params(),
      out_type=jax.ShapeDtypeStruct((B,), jnp.float32),
      mesh=_mesh(),
      scratch_types=[
          pltpu.VMEM((B_PER_W,), jnp.int32),
          pltpu.VMEM((B_PER_W,), jnp.int32),
          pltpu.VMEM((B_PER_W,), jnp.float32),
          pltpu.VMEM((B_PER_W,), jnp.float32),
          pltpu.VMEM((B_PER_W,), jnp.float32),
          pltpu.SemaphoreType.DMA,
      ],
  )
  def k(word_u_hbm, word_v_hbm, in_bias_hbm, out_bias_hbm, colsum_hbm,
        idx_u_v, idx_v_v, ub, vb, colsum_v, sem):
    wid = lax.axis_index("s") * 2 + lax.axis_index("c")
    base = wid * B_PER_W

    c1 = pltpu.async_copy(word_u_hbm.at[pl.ds(base, B_PER_W)], idx_u_v, sem)
    c2 = pltpu.async_copy(word_v_hbm.at[pl.ds(base, B_PER_W)], idx_v_v, sem)
    c1.wait()
    c2.wait()
    g1 = pltpu.async_copy(in_bias_hbm.at[idx_u_v], ub, sem)
    g2 = pltpu.async_copy(out_bias_hbm.at[idx_v_v], vb, sem)
    g1.wait()
    g2.wait()
    for g in range(B_PER_W // LANES):
      colsum_v[pl.ds(g * LANES, LANES)] = (
          ub[pl.ds(g * LANES, LANES)] + vb[pl.ds(g * LANES, LANES)])
    s1 = pltpu.async_copy(colsum_v, colsum_hbm.at[pl.ds(base, B_PER_W)], sem)
    s1.wait()

  return k(word_u, word_v, in_bias_flat, out_bias_flat)


_COL_BLK = 512


def _tc_broadcast_body(colsum_ref, dots_ref, out_ref):
  out_ref[...] = colsum_ref[...] + dots_ref[...]


def _tc_broadcast(colsum_col, dots_row):
  return pl.pallas_call(
      _tc_broadcast_body,
      grid=(B // _COL_BLK,),
      in_specs=[
          pl.BlockSpec((B, 1), lambda j: (0, 0)),
          pl.BlockSpec((1, _COL_BLK), lambda j: (0, j)),
      ],
      out_specs=pl.BlockSpec((B, _COL_BLK), lambda j: (0, j)),
      out_shape=jax.ShapeDtypeStruct((B, B), jnp.float32),
  )(colsum_col, dots_row)


def kernel(word_u, word_v, in_embed, in_bias, out_embed, out_bias):
  wu = word_u.astype(jnp.int32)
  wv = word_v.astype(jnp.int32)
  dots = _sc_embed_dot(wu, wv, in_embed.T, out_embed.T)
  colsum = _sc_bias_sum(wu, wv, in_bias.reshape(-1), out_bias.reshape(-1))
  return _tc_broadcast(colsum.reshape(B, 1), dots.reshape(1, B))


# speedup vs baseline: 1.4663x; 1.4663x over previous
import dataclasses
import functools

import jax
import jax.numpy as jnp
from jax import lax
from jax.experimental import pallas as pl
from jax.experimental.pallas import tpu as pltpu
from jax.experimental.pallas import tpu_sc as plsc

B = 4096
D = 32
NUM_WORKERS = 32
B_PER_W = B // NUM_WORKERS
LANES = 16
W = 128
CH = 16
N_CH = B_PER_W // CH


def _sc_params():
  cp = pltpu.CompilerParams()
  if "needs_layout_passes" in pltpu.CompilerParams.__dataclass_fields__:
    cp = dataclasses.replace(cp, needs_layout_passes=False)
  if "use_tc_tiling_on_sc" in pltpu.CompilerParams.__dataclass_fields__:
    cp = dataclasses.replace(cp, use_tc_tiling_on_sc=True)
  return cp


def _mesh():
  return plsc.VectorSubcoreMesh(core_axis_name="c", subcore_axis_name="s")


def _sc_embed_dot(word_u, word_v, in_embed_t, out_embed_t):
  @functools.partial(
      pl.kernel,
      compiler_params=_sc_params(),
      out_type=jax.ShapeDtypeStruct((B,), jnp.float32),
      mesh=_mesh(),
      scratch_types=[
          pltpu.VMEM((B_PER_W,), jnp.int32),
          pltpu.VMEM((B_PER_W,), jnp.int32),
          pltpu.VMEM((CH, D, W), jnp.float32),
          pltpu.VMEM((D, CH), jnp.float32),
          pltpu.VMEM((B_PER_W,), jnp.float32),
          pltpu.SemaphoreType.DMA,
          pltpu.SemaphoreType.DMA,
      ],
  )
  def k(word_u_hbm, word_v_hbm, u_tab_hbm, v_tab_hbm, dots_hbm,
        idx_u_v, idx_v_v, blk, u_comp, dots_v, sem, bsem):
    wid = lax.axis_index("s") * 2 + lax.axis_index("c")
    base = wid * B_PER_W

    c1 = pltpu.async_copy(word_u_hbm.at[pl.ds(base, B_PER_W)], idx_u_v, bsem)
    c2 = pltpu.async_copy(word_v_hbm.at[pl.ds(base, B_PER_W)], idx_v_v, bsem)
    c1.wait()
    c2.wait()

    iota = lax.iota(jnp.int32, LANES)
    for c in range(N_CH):
      off = c * CH
      lu = idx_u_v[pl.ds(off, LANES)]
      lv = idx_v_v[pl.ds(off, LANES)]
      jb_u = (lu // W) * W
      jb_v = (lv // W) * W
      cops = []
      for i in range(CH):
        ju = jnp.sum(jnp.where(iota == i, jb_u, 0))
        cops.append(pltpu.async_copy(
            u_tab_hbm.at[:, pl.ds(pl.multiple_of(ju, W), W)], blk.at[i], sem))
      for cp_ in cops:
        cp_.wait()
      lane_u = lu - jb_u
      for d in range(D):
        dd = jnp.full((LANES,), d, jnp.int32)
        u_comp[d, :] = plsc.load_gather(blk, [iota, dd, lane_u])
      cops = []
      for i in range(CH):
        jv = jnp.sum(jnp.where(iota == i, jb_v, 0))
        cops.append(pltpu.async_copy(
            v_tab_hbm.at[:, pl.ds(pl.multiple_of(jv, W), W)], blk.at[i], sem))
      for cp_ in cops:
        cp_.wait()
      lane_v = lv - jb_v
      acc = jnp.zeros((LANES,), jnp.float32)
      for d in range(D):
        dd = jnp.full((LANES,), d, jnp.int32)
        vv = plsc.load_gather(blk, [iota, dd, lane_v])
        acc = acc + u_comp[d, :] * vv
      dots_v[pl.ds(off, LANES)] = acc

    s1 = pltpu.async_copy(dots_v, dots_hbm.at[pl.ds(base, B_PER_W)], bsem)
    s1.wait()

  return k(word_u, word_v, in_embed_t, out_embed_t)


def _sc_bias_sum(word_u, word_v, in_bias_flat, out_bias_flat):
  @functools.partial(
      pl.kernel,
      compiler_params=_sc_params(),
      out_type=jax.ShapeDtypeStruct((B,), jnp.float32),
      mesh=_mesh(),
      scratch_types=[
          pltpu.VMEM((B_PER_W,), jnp.int32),
          pltpu.VMEM((B_PER_W,), jnp.int32),
          pltpu.VMEM((B_PER_W,), jnp.float32),
          pltpu.VMEM((B_PER_W,), jnp.float32),
          pltpu.VMEM((B_PER_W,), jnp.float32),
          pltpu.SemaphoreType.DMA,
      ],
  )
  def k(word_u_hbm, word_v_hbm, in_bias_hbm, out_bias_hbm, colsum_hbm,
        idx_u_v, idx_v_v, ub, vb, colsum_v, sem):
    wid = lax.axis_index("s") * 2 + lax.axis_index("c")
    base = wid * B_PER_W

    c1 = pltpu.async_copy(word_u_hbm.at[pl.ds(base, B_PER_W)], idx_u_v, sem)
    c2 = pltpu.async_copy(word_v_hbm.at[pl.ds(base, B_PER_W)], idx_v_v, sem)
    c1.wait()
    c2.wait()
    g1 = pltpu.async_copy(in_bias_hbm.at[idx_u_v], ub, sem)
    g2 = pltpu.async_copy(out_bias_hbm.at[idx_v_v], vb, sem)
    g1.wait()
    g2.wait()
    for g in range(B_PER_W // LANES):
      colsum_v[pl.ds(g * LANES, LANES)] = (
          ub[pl.ds(g * LANES, LANES)] + vb[pl.ds(g * LANES, LANES)])
    s1 = pltpu.async_copy(colsum_v, colsum_hbm.at[pl.ds(base, B_PER_W)], sem)
    s1.wait()

  return k(word_u, word_v, in_bias_flat, out_bias_flat)


_COL_BLK = 512


def _tc_broadcast_body(colsum_ref, dots_ref, out_ref):
  out_ref[...] = colsum_ref[...] + dots_ref[...]


def _tc_broadcast(colsum_col, dots_row):
  return pl.pallas_call(
      _tc_broadcast_body,
      grid=(B // _COL_BLK,),
      in_specs=[
          pl.BlockSpec((B, 1), lambda j: (0, 0)),
          pl.BlockSpec((1, _COL_BLK), lambda j: (0, j)),
      ],
      out_specs=pl.BlockSpec((B, _COL_BLK), lambda j: (0, j)),
      out_shape=jax.ShapeDtypeStruct((B, B), jnp.float32),
  )(colsum_col, dots_row)


def kernel(word_u, word_v, in_embed, in_bias, out_embed, out_bias):
  wu = word_u.astype(jnp.int32)
  wv = word_v.astype(jnp.int32)
  dots = _sc_embed_dot(wu, wv, in_embed.T, out_embed.T)
  wu_b, _ = lax.optimization_barrier((wu, dots))
  colsum = _sc_bias_sum(wu_b, wv, in_bias.T.reshape(-1),
                        out_bias.T.reshape(-1))
  return _tc_broadcast(colsum.reshape(B, 1), dots.reshape(1, B))


# speedup vs baseline: 1.5817x; 1.0787x over previous
import dataclasses
import functools

import jax
import jax.numpy as jnp
from jax import lax
from jax.experimental import pallas as pl
from jax.experimental.pallas import tpu as pltpu
from jax.experimental.pallas import tpu_sc as plsc

B = 4096
D = 32
NUM_WORKERS = 32
B_PER_W = B // NUM_WORKERS
LANES = 16
W = 128
CH = 16
N_CH = B_PER_W // CH


def _sc_params():
  cp = pltpu.CompilerParams()
  if "needs_layout_passes" in pltpu.CompilerParams.__dataclass_fields__:
    cp = dataclasses.replace(cp, needs_layout_passes=False)
  if "use_tc_tiling_on_sc" in pltpu.CompilerParams.__dataclass_fields__:
    cp = dataclasses.replace(cp, use_tc_tiling_on_sc=True)
  return cp


def _mesh():
  return plsc.VectorSubcoreMesh(core_axis_name="c", subcore_axis_name="s")


def _sc_embed_dot(word_u, word_v, in_embed_t, out_embed_t):
  @functools.partial(
      pl.kernel,
      compiler_params=_sc_params(),
      out_type=jax.ShapeDtypeStruct((B,), jnp.float32),
      mesh=_mesh(),
      scratch_types=[
          pltpu.VMEM((B_PER_W,), jnp.int32),
          pltpu.VMEM((B_PER_W,), jnp.int32),
          pltpu.VMEM((CH, D, W), jnp.float32),
          pltpu.VMEM((D, CH), jnp.float32),
          pltpu.VMEM((B_PER_W,), jnp.float32),
          pltpu.SemaphoreType.DMA,
          pltpu.SemaphoreType.DMA,
      ],
  )
  def k(word_u_hbm, word_v_hbm, u_tab_hbm, v_tab_hbm, dots_hbm,
        idx_u_v, idx_v_v, blk, u_comp, dots_v, sem, bsem):
    wid = lax.axis_index("s") * 2 + lax.axis_index("c")
    base = wid * B_PER_W

    c1 = pltpu.async_copy(word_u_hbm.at[pl.ds(base, B_PER_W)], idx_u_v, bsem)
    c2 = pltpu.async_copy(word_v_hbm.at[pl.ds(base, B_PER_W)], idx_v_v, bsem)
    c1.wait()
    c2.wait()

    iota = lax.iota(jnp.int32, LANES)
    for c in range(N_CH):
      off = c * CH
      lu = idx_u_v[pl.ds(off, LANES)]
      lv = idx_v_v[pl.ds(off, LANES)]
      jb_u = (lu // W) * W
      jb_v = (lv // W) * W
      cops = []
      for i in range(CH):
        ju = jnp.sum(jnp.where(iota == i, jb_u, 0))
        cops.append(pltpu.async_copy(
            u_tab_hbm.at[:, pl.ds(pl.multiple_of(ju, W), W)], blk.at[i], sem))
      for cp_ in cops:
        cp_.wait()
      lane_u = lu - jb_u
      for d in range(D):
        dd = jnp.full((LANES,), d, jnp.int32)
        u_comp[d, :] = plsc.load_gather(blk, [iota, dd, lane_u])
      cops = []
      for i in range(CH):
        jv = jnp.sum(jnp.where(iota == i, jb_v, 0))
        cops.append(pltpu.async_copy(
            v_tab_hbm.at[:, pl.ds(pl.multiple_of(jv, W), W)], blk.at[i], sem))
      for cp_ in cops:
        cp_.wait()
      lane_v = lv - jb_v
      acc = jnp.zeros((LANES,), jnp.float32)
      for d in range(D):
        dd = jnp.full((LANES,), d, jnp.int32)
        vv = plsc.load_gather(blk, [iota, dd, lane_v])
        acc = acc + u_comp[d, :] * vv
      dots_v[pl.ds(off, LANES)] = acc

    s1 = pltpu.async_copy(dots_v, dots_hbm.at[pl.ds(base, B_PER_W)], bsem)
    s1.wait()

  return k(word_u, word_v, in_embed_t, out_embed_t)


_BR = 1
_BC = 1000000 // _BR


def _sc_bias_sum(word_u, word_v, in_bias_2d, out_bias_2d):
  @functools.partial(
      pl.kernel,
      compiler_params=_sc_params(),
      out_type=jax.ShapeDtypeStruct((B,), jnp.float32),
      mesh=_mesh(),
      scratch_types=[
          pltpu.VMEM((B_PER_W,), jnp.int32),
          pltpu.VMEM((B_PER_W,), jnp.int32),
          pltpu.VMEM((LANES, _BR, 128), jnp.float32),
          pltpu.VMEM((LANES, _BR, 128), jnp.float32),
          pltpu.VMEM((B_PER_W,), jnp.float32),
          pltpu.SemaphoreType.DMA,
      ],
  )
  def k(word_u_hbm, word_v_hbm, in_bias_hbm, out_bias_hbm, colsum_hbm,
        idx_u_v, idx_v_v, blk_u, blk_v, colsum_v, sem):
    wid = lax.axis_index("s") * 2 + lax.axis_index("c")
    base = wid * B_PER_W

    c1 = pltpu.async_copy(word_u_hbm.at[pl.ds(base, B_PER_W)], idx_u_v, sem)
    c2 = pltpu.async_copy(word_v_hbm.at[pl.ds(base, B_PER_W)], idx_v_v, sem)
    c1.wait()
    c2.wait()

    iota = lax.iota(jnp.int32, LANES)
    zero = jnp.zeros((LANES,), jnp.int32)
    for g in range(B_PER_W // LANES):
      off = g * LANES
      ju = idx_u_v[pl.ds(off, LANES)]
      jv = idx_v_v[pl.ds(off, LANES)]
      cb_u = (ju // 128) * 128
      cb_v = (jv // 128) * 128
      cops = []
      for i in range(LANES):
        su = jnp.sum(jnp.where(iota == i, cb_u, 0))
        sv = jnp.sum(jnp.where(iota == i, cb_v, 0))
        cops.append(pltpu.async_copy(
            in_bias_hbm.at[:, pl.ds(pl.multiple_of(su, 128), 128)],
            blk_u.at[i], sem))
        cops.append(pltpu.async_copy(
            out_bias_hbm.at[:, pl.ds(pl.multiple_of(sv, 128), 128)],
            blk_v.at[i], sem))
      for cp_ in cops:
        cp_.wait()
      ubv = plsc.load_gather(blk_u, [iota, zero, ju - cb_u])
      vbv = plsc.load_gather(blk_v, [iota, zero, jv - cb_v])
      colsum_v[pl.ds(off, LANES)] = ubv + vbv

    s1 = pltpu.async_copy(colsum_v, colsum_hbm.at[pl.ds(base, B_PER_W)], sem)
    s1.wait()

  return k(word_u, word_v, in_bias_2d, out_bias_2d)


_COL_BLK = 512


def _tc_broadcast_body(colsum_ref, dots_ref, out_ref):
  out_ref[...] = colsum_ref[...] + dots_ref[...]


def _tc_broadcast(colsum_col, dots_row):
  return pl.pallas_call(
      _tc_broadcast_body,
      grid=(B // _COL_BLK,),
      in_specs=[
          pl.BlockSpec((B, 1), lambda j: (0, 0)),
          pl.BlockSpec((1, _COL_BLK), lambda j: (0, j)),
      ],
      out_specs=pl.BlockSpec((B, _COL_BLK), lambda j: (0, j)),
      out_shape=jax.ShapeDtypeStruct((B, B), jnp.float32),
  )(colsum_col, dots_row)


def kernel(word_u, word_v, in_embed, in_bias, out_embed, out_bias):
  wu = word_u.astype(jnp.int32)
  wv = word_v.astype(jnp.int32)
  dots = _sc_embed_dot(wu, wv, in_embed.T, out_embed.T)
  wu_b, _ = lax.optimization_barrier((wu, dots))
  colsum = _sc_bias_sum(wu_b, wv, in_bias.T, out_bias.T)
  return _tc_broadcast(colsum.reshape(B, 1), dots.reshape(1, B))


# speedup vs baseline: 1.7539x; 1.1089x over previous
import dataclasses
import functools

import jax
import jax.numpy as jnp
from jax import lax
from jax.experimental import pallas as pl
from jax.experimental.pallas import tpu as pltpu
from jax.experimental.pallas import tpu_sc as plsc

B = 4096
D = 32
NUM_WORKERS = 32
B_PER_W = B // NUM_WORKERS
LANES = 16
W = 128
CH = 16
N_CH = B_PER_W // CH


def _sc_params():
  cp = pltpu.CompilerParams()
  if "needs_layout_passes" in pltpu.CompilerParams.__dataclass_fields__:
    cp = dataclasses.replace(cp, needs_layout_passes=False)
  if "use_tc_tiling_on_sc" in pltpu.CompilerParams.__dataclass_fields__:
    cp = dataclasses.replace(cp, use_tc_tiling_on_sc=True)
  return cp


def _sc_gather_dot(word_u, word_v, in_embed_t, out_embed_t, in_bias_t,
                   out_bias_t):
  mesh = plsc.VectorSubcoreMesh(core_axis_name="c", subcore_axis_name="s")

  @functools.partial(
      pl.kernel,
      compiler_params=_sc_params(),
      out_type=(
          jax.ShapeDtypeStruct((B,), jnp.float32),
          jax.ShapeDtypeStruct((B,), jnp.float32),
      ),
      mesh=mesh,
      scratch_types=[
          pltpu.VMEM((B_PER_W,), jnp.int32),
          pltpu.VMEM((B_PER_W,), jnp.int32),
          pltpu.VMEM((CH, D, W), jnp.float32),
          pltpu.VMEM((CH, 1, W), jnp.float32),
          pltpu.VMEM((D, CH), jnp.float32),
          pltpu.VMEM((B_PER_W,), jnp.float32),
          pltpu.VMEM((B_PER_W,), jnp.float32),
          pltpu.SemaphoreType.DMA,
          pltpu.SemaphoreType.DMA,
      ],
  )
  def k(word_u_hbm, word_v_hbm, u_tab_hbm, v_tab_hbm, ub_hbm, vb_hbm,
        dots_hbm, colsum_hbm, idx_u_v, idx_v_v, blk, bblk, u_comp, dots_v,
        colsum_v, sem, bsem):
    wid = lax.axis_index("s") * 2 + lax.axis_index("c")
    base = wid * B_PER_W

    c1 = pltpu.async_copy(word_u_hbm.at[pl.ds(base, B_PER_W)], idx_u_v, bsem)
    c2 = pltpu.async_copy(word_v_hbm.at[pl.ds(base, B_PER_W)], idx_v_v, bsem)
    c1.wait()
    c2.wait()

    iota = lax.iota(jnp.int32, LANES)
    zero = jnp.zeros((LANES,), jnp.int32)
    for c in range(N_CH):
      off = c * CH
      lu = idx_u_v[pl.ds(off, LANES)]
      lv = idx_v_v[pl.ds(off, LANES)]
      jb_u = (lu // W) * W
      jb_v = (lv // W) * W
      cops = []
      for i in range(CH):
        ju = pl.multiple_of(jnp.sum(jnp.where(iota == i, jb_u, 0)), W)
        cops.append(pltpu.async_copy(
            u_tab_hbm.at[:, pl.ds(ju, W)], blk.at[i], sem))
        cops.append(pltpu.async_copy(
            ub_hbm.at[:, pl.ds(ju, W)], bblk.at[i], bsem))
      for cp_ in cops:
        cp_.wait()
      lane_u = lu - jb_u
      for d in range(D):
        dd = jnp.full((LANES,), d, jnp.int32)
        u_comp[d, :] = plsc.load_gather(blk, [iota, dd, lane_u])
      ub16 = plsc.load_gather(bblk, [iota, zero, lane_u])
      cops = []
      for i in range(CH):
        jv = pl.multiple_of(jnp.sum(jnp.where(iota == i, jb_v, 0)), W)
        cops.append(pltpu.async_copy(
            v_tab_hbm.at[:, pl.ds(jv, W)], blk.at[i], sem))
        cops.append(pltpu.async_copy(
            vb_hbm.at[:, pl.ds(jv, W)], bblk.at[i], bsem))
      for cp_ in cops:
        cp_.wait()
      lane_v = lv - jb_v
      acc = jnp.zeros((LANES,), jnp.float32)
      for d in range(D):
        dd = jnp.full((LANES,), d, jnp.int32)
        vv = plsc.load_gather(blk, [iota, dd, lane_v])
        acc = acc + u_comp[d, :] * vv
      dots_v[pl.ds(off, LANES)] = acc
      vb16 = plsc.load_gather(bblk, [iota, zero, lane_v])
      colsum_v[pl.ds(off, LANES)] = ub16 + vb16

    s1 = pltpu.async_copy(dots_v, dots_hbm.at[pl.ds(base, B_PER_W)], bsem)
    s2 = pltpu.async_copy(colsum_v, colsum_hbm.at[pl.ds(base, B_PER_W)], bsem)
    s1.wait()
    s2.wait()

  return k(word_u, word_v, in_embed_t, out_embed_t, in_bias_t, out_bias_t)


_COL_BLK = 512


def _tc_broadcast_body(colsum_ref, dots_ref, out_ref):
  out_ref[...] = colsum_ref[...] + dots_ref[...]


def _tc_broadcast(colsum_col, dots_row):
  return pl.pallas_call(
      _tc_broadcast_body,
      grid=(B // _COL_BLK,),
      in_specs=[
          pl.BlockSpec((B, 1), lambda j: (0, 0)),
          pl.BlockSpec((1, _COL_BLK), lambda j: (0, j)),
      ],
      out_specs=pl.BlockSpec((B, _COL_BLK), lambda j: (0, j)),
      out_shape=jax.ShapeDtypeStruct((B, B), jnp.float32),
  )(colsum_col, dots_row)


def kernel(word_u, word_v, in_embed, in_bias, out_embed, out_bias):
  wu = word_u.astype(jnp.int32)
  wv = word_v.astype(jnp.int32)
  dots, colsum = _sc_gather_dot(wu, wv, in_embed.T, out_embed.T,
                                in_bias.T, out_bias.T)
  return _tc_broadcast(colsum.reshape(B, 1), dots.reshape(1, B))
